# baseline (speedup 1.0000x reference)
import jax
import jax.numpy as jnp
from jax import lax
from jax.experimental import pallas as pl
from jax.experimental.pallas import tpu as pltpu
from jax.experimental.pallas import tpu_sc as plsc

H = 128
DEPTH = 4
NC, NS = 2, 16
NW = NC * NS

NB = 320000
NA = 10000
NAP = NW * 320
MAX_NB = 32

GS_CHUNKS = 80
ED_CHUNK = 80
ED_CHUNKS = 125


def _init_mm_kernel(fb_ref, wi_ref, out_ref):
    out_ref[...] = jnp.maximum(
        jnp.dot(fb_ref[...], wi_ref[...], preferred_element_type=jnp.float32),
        0.0)


def _layer_mm_kernel(fb_ref, t_ref, wi_ref, wh_ref, out_ref):
    acc = jnp.dot(fb_ref[...], wi_ref[...], preferred_element_type=jnp.float32)
    acc = acc + jnp.dot(t_ref[...], wh_ref[...],
                        preferred_element_type=jnp.float32)
    out_ref[...] = jnp.maximum(acc, 0.0)


def _final_mm_kernel(fa_ref, am_ref, wo1_ref, wo2_ref, mask_ref, out_ref):
    acc = jnp.dot(fa_ref[...], wo1_ref[...], preferred_element_type=jnp.float32)
    acc = acc + jnp.dot(am_ref[...], wo2_ref[...],
                        preferred_element_type=jnp.float32)
    out_ref[...] = jnp.maximum(acc, 0.0) * mask_ref[...]


def _init_mm(f_bonds, W_i, br=3200):
    nb, k = f_bonds.shape
    return pl.pallas_call(
        _init_mm_kernel,
        grid=(nb // br,),
        in_specs=[
            pl.BlockSpec((br, k), lambda i: (i, 0)),
            pl.BlockSpec((k, H), lambda i: (0, 0)),
        ],
        out_specs=pl.BlockSpec((br, H), lambda i: (i, 0)),
        out_shape=jax.ShapeDtypeStruct((nb, H), jnp.float32),
    )(f_bonds, W_i)


def _layer_mm(f_bonds, t, W_i, W_h, br=3200):
    nb, k = f_bonds.shape
    return pl.pallas_call(
        _layer_mm_kernel,
        grid=(nb // br,),
        in_specs=[
            pl.BlockSpec((br, k), lambda i: (i, 0)),
            pl.BlockSpec((br, H), lambda i: (i, 0)),
            pl.BlockSpec((k, H), lambda i: (0, 0)),
            pl.BlockSpec((H, H), lambda i: (0, 0)),
        ],
        out_specs=pl.BlockSpec((br, H), lambda i: (i, 0)),
        out_shape=jax.ShapeDtypeStruct((nb, H), jnp.float32),
    )(f_bonds, t, W_i, W_h)


def _final_mm(f_atoms, a_msg, W_o, mask, br=2000):
    na, fd = f_atoms.shape
    return pl.pallas_call(
        _final_mm_kernel,
        grid=(na // br,),
        in_specs=[
            pl.BlockSpec((br, fd), lambda i: (i, 0)),
            pl.BlockSpec((br, H), lambda i: (i, 0)),
            pl.BlockSpec((fd, H), lambda i: (0, 0)),
            pl.BlockSpec((H, H), lambda i: (0, 0)),
            pl.BlockSpec((br, 1), lambda i: (i, 0)),
        ],
        out_specs=pl.BlockSpec((br, H), lambda i: (i, 0)),
        out_shape=jax.ShapeDtypeStruct((na, H), jnp.float32),
    )(f_atoms, a_msg, W_o[:fd], W_o[fd:], mask)


def _gs_body(msg_hbm, a2b_hbm, amsg_hbm, idx_v, rows_v, out_v, sem0, sem1):
    wid = lax.axis_index("s") * NC + lax.axis_index("c")
    pltpu.sync_copy(a2b_hbm.at[wid], idx_v)
    sems = (sem0, sem1)
    pltpu.async_copy(msg_hbm.at[idx_v.at[0]], rows_v.at[0], sem0)
    pltpu.async_copy(msg_hbm.at[idx_v.at[1]], rows_v.at[1], sem1)

    def step(s, carry):
        for b in (0, 1):
            c = 2 * s + b
            pltpu.make_async_copy(
                msg_hbm.at[idx_v.at[c]], rows_v.at[b], sems[b]).wait()
            for i in range(4):
                for j in range(8):
                    acc = rows_v[b, 32 * i, pl.ds(16 * j, 16)]
                    for k in range(1, 32):
                        acc = acc + rows_v[b, 32 * i + k, pl.ds(16 * j, 16)]
                    out_v[pl.ds((4 * c + i) * H + 16 * j, 16)] = acc
            pltpu.async_copy(msg_hbm.at[idx_v.at[c + 2]], rows_v.at[b], sems[b])
        return carry

    lax.fori_loop(0, GS_CHUNKS // 2, step, 0)
    pltpu.make_async_copy(msg_hbm.at[idx_v.at[0]], rows_v.at[0], sem0).wait()
    pltpu.make_async_copy(msg_hbm.at[idx_v.at[1]], rows_v.at[1], sem1).wait()
    pltpu.sync_copy(out_v, amsg_hbm.at[pl.ds(wid * 320 * H, 320 * H)])


def _sc_gather_sum(message, a2b_r):
    out_flat = pl.kernel(
        _gs_body,
        out_type=jax.ShapeDtypeStruct((NAP * H,), jnp.float32),
        mesh=plsc.VectorSubcoreMesh(core_axis_name="c", subcore_axis_name="s"),
        scratch_types=[
            pltpu.VMEM((GS_CHUNKS + 2, 128), jnp.int32),
            pltpu.VMEM((2, 128, H), jnp.float32),
            pltpu.VMEM((320 * H,), jnp.float32),
            pltpu.SemaphoreType.DMA,
            pltpu.SemaphoreType.DMA,
        ],
    )(message, a2b_r)
    return out_flat.reshape(NAP, H)


def _edge_body(amsg_hbm, msg_hbm, b2a_hbm, b2r_hbm, t_hbm, dummy_hbm,
               idxa_v, idxr_v, ga_v, gr_v, to_v,
               sa0, sa1, sr0, sr1, so0, so1):
    wid = lax.axis_index("s") * NC + lax.axis_index("c")
    pltpu.sync_copy(b2a_hbm.at[wid], idxa_v)
    pltpu.sync_copy(b2r_hbm.at[wid], idxr_v)
    sas, srs, sos = (sa0, sa1), (sr0, sr1), (so0, so1)
    base = wid * 10000
    for b in (0, 1):
        pltpu.async_copy(amsg_hbm.at[idxa_v.at[b]], ga_v.at[b], sas[b])
        pltpu.async_copy(msg_hbm.at[idxr_v.at[b]], gr_v.at[b], srs[b])
        pltpu.async_copy(to_v.at[b], dummy_hbm.at[wid], sos[b])

    def chunk(c, b):
        pltpu.make_async_copy(
            amsg_hbm.at[idxa_v.at[c]], ga_v.at[b], sas[b]).wait()
        pltpu.make_async_copy(
            msg_hbm.at[idxr_v.at[c]], gr_v.at[b], srs[b]).wait()
        pltpu.make_async_copy(to_v.at[b], dummy_hbm.at[wid], sos[b]).wait()
        for r in range(ED_CHUNK):
            for j in range(8):
                to_v[b, r, pl.ds(16 * j, 16)] = (
                    ga_v[b, r, pl.ds(16 * j, 16)]
                    - gr_v[b, r, pl.ds(16 * j, 16)])
        pltpu.async_copy(
            to_v.at[b], t_hbm.at[pl.ds(base + c * ED_CHUNK, ED_CHUNK)], sos[b])
        pltpu.async_copy(amsg_hbm.at[idxa_v.at[c + 2]], ga_v.at[b], sas[b])
        pltpu.async_copy(msg_hbm.at[idxr_v.at[c + 2]], gr_v.at[b], srs[b])

    def step(s, carry):
        chunk(2 * s, 0)
        chunk(2 * s + 1, 1)
        return carry

    lax.fori_loop(0, (ED_CHUNKS - 1) // 2, step, 0)
    chunk(ED_CHUNKS - 1, 0)
    pltpu.make_async_copy(amsg_hbm.at[idxa_v.at[0]], ga_v.at[0], sa0).wait()
    pltpu.make_async_copy(msg_hbm.at[idxr_v.at[0]], gr_v.at[0], sr0).wait()
    pltpu.make_async_copy(amsg_hbm.at[idxa_v.at[1]], ga_v.at[1], sa1).wait()
    pltpu.make_async_copy(msg_hbm.at[idxr_v.at[1]], gr_v.at[1], sr1).wait()
    pltpu.make_async_copy(to_v.at[0], dummy_hbm.at[wid], so0).wait()
    pltpu.make_async_copy(to_v.at[1], dummy_hbm.at[wid], so1).wait()


def _sc_edge(a_msg, message, b2a_r, b2r_r):
    t, _ = pl.kernel(
        _edge_body,
        out_type=[
            jax.ShapeDtypeStruct((NB, H), jnp.float32),
            jax.ShapeDtypeStruct((NW, ED_CHUNK, H), jnp.float32),
        ],
        mesh=plsc.VectorSubcoreMesh(core_axis_name="c", subcore_axis_name="s"),
        scratch_types=[
            pltpu.VMEM((ED_CHUNKS + 3, ED_CHUNK), jnp.int32),
            pltpu.VMEM((ED_CHUNKS + 3, ED_CHUNK), jnp.int32),
            pltpu.VMEM((2, ED_CHUNK, H), jnp.float32),
            pltpu.VMEM((2, ED_CHUNK, H), jnp.float32),
            pltpu.VMEM((2, ED_CHUNK, H), jnp.float32),
            pltpu.SemaphoreType.DMA,
            pltpu.SemaphoreType.DMA,
            pltpu.SemaphoreType.DMA,
            pltpu.SemaphoreType.DMA,
            pltpu.SemaphoreType.DMA,
            pltpu.SemaphoreType.DMA,
        ],
    )(a_msg, message, b2a_r, b2r_r)
    return t


def kernel(f_atoms, f_bonds, a2b, b2a, b2revb, mask, W_i, W_h, W_o):
    a2b = a2b.astype(jnp.int32)
    b2a = b2a.astype(jnp.int32)
    b2revb = b2revb.astype(jnp.int32)

    a2b_pad = jnp.zeros((NAP, MAX_NB), jnp.int32).at[:NA].set(a2b)
    a2b_r = jnp.pad(a2b_pad.reshape(NW, GS_CHUNKS, 128), ((0, 0), (0, 2), (0, 0)))
    b2a_r = jnp.pad(b2a.reshape(NW, ED_CHUNKS, ED_CHUNK), ((0, 0), (0, 3), (0, 0)))
    b2r_r = jnp.pad(b2revb.reshape(NW, ED_CHUNKS, ED_CHUNK), ((0, 0), (0, 3), (0, 0)))

    message = _init_mm(f_bonds, W_i)
    for _ in range(DEPTH - 1):
        a_msg = _sc_gather_sum(message, a2b_r)
        t = _sc_edge(a_msg, message, b2a_r, b2r_r)
        message = _layer_mm(f_bonds, t, W_i, W_h)
    a_msg = _sc_gather_sum(message, a2b_r)
    return _final_mm(f_atoms, a_msg[:NA], W_o, mask)


# speedup vs baseline: 1.0132x; 1.0132x over previous
import jax
import jax.numpy as jnp
from jax import lax
from jax.experimental import pallas as pl
from jax.experimental.pallas import tpu as pltpu
from jax.experimental.pallas import tpu_sc as plsc

H = 128
DEPTH = 4
NC, NS = 2, 16
NW = NC * NS

NB = 320000
NA = 10000
NAP = NW * 320
MAX_NB = 32

GS_CHUNK = 64
GS_CHUNKS = 160
GS_RING = 4
ED_CHUNK = 40
ED_CHUNKS = 250
ED_RING = 5


def _init_mm_kernel(fb_ref, wi_ref, out_ref):
    out_ref[...] = jnp.maximum(
        jnp.dot(fb_ref[...], wi_ref[...], preferred_element_type=jnp.float32),
        0.0)


def _layer_mm_kernel(fb_ref, t_ref, wi_ref, wh_ref, out_ref):
    acc = jnp.dot(fb_ref[...], wi_ref[...], preferred_element_type=jnp.float32)
    acc = acc + jnp.dot(t_ref[...], wh_ref[...],
                        preferred_element_type=jnp.float32)
    out_ref[...] = jnp.maximum(acc, 0.0)


def _final_mm_kernel(fa_ref, am_ref, wo1_ref, wo2_ref, mask_ref, out_ref):
    acc = jnp.dot(fa_ref[...], wo1_ref[...], preferred_element_type=jnp.float32)
    acc = acc + jnp.dot(am_ref[...], wo2_ref[...],
                        preferred_element_type=jnp.float32)
    out_ref[...] = jnp.maximum(acc, 0.0) * mask_ref[...]


def _init_mm(f_bonds, W_i, br=3200):
    nb, k = f_bonds.shape
    return pl.pallas_call(
        _init_mm_kernel,
        grid=(nb // br,),
        in_specs=[
            pl.BlockSpec((br, k), lambda i: (i, 0)),
            pl.BlockSpec((k, H), lambda i: (0, 0)),
        ],
        out_specs=pl.BlockSpec((br, H), lambda i: (i, 0)),
        out_shape=jax.ShapeDtypeStruct((nb, H), jnp.float32),
    )(f_bonds, W_i)


def _layer_mm(f_bonds, t, W_i, W_h, br=3200):
    nb, k = f_bonds.shape
    return pl.pallas_call(
        _layer_mm_kernel,
        grid=(nb // br,),
        in_specs=[
            pl.BlockSpec((br, k), lambda i: (i, 0)),
            pl.BlockSpec((br, H), lambda i: (i, 0)),
            pl.BlockSpec((k, H), lambda i: (0, 0)),
            pl.BlockSpec((H, H), lambda i: (0, 0)),
        ],
        out_specs=pl.BlockSpec((br, H), lambda i: (i, 0)),
        out_shape=jax.ShapeDtypeStruct((nb, H), jnp.float32),
    )(f_bonds, t, W_i, W_h)


def _final_mm(f_atoms, a_msg, W_o, mask, br=2000):
    na, fd = f_atoms.shape
    return pl.pallas_call(
        _final_mm_kernel,
        grid=(na // br,),
        in_specs=[
            pl.BlockSpec((br, fd), lambda i: (i, 0)),
            pl.BlockSpec((br, H), lambda i: (i, 0)),
            pl.BlockSpec((fd, H), lambda i: (0, 0)),
            pl.BlockSpec((H, H), lambda i: (0, 0)),
            pl.BlockSpec((br, 1), lambda i: (i, 0)),
        ],
        out_specs=pl.BlockSpec((br, H), lambda i: (i, 0)),
        out_shape=jax.ShapeDtypeStruct((na, H), jnp.float32),
    )(f_atoms, a_msg, W_o[:fd], W_o[fd:], mask)


def _gs_body(msg_hbm, a2b_hbm, amsg_hbm, idx_v, rows_v, out_v, *sems):
    wid = lax.axis_index("s") * NC + lax.axis_index("c")
    pltpu.sync_copy(a2b_hbm.at[wid], idx_v)

    def gidx(c):
        return idx_v.at[pl.ds(c * GS_CHUNK, GS_CHUNK)]

    for b in range(GS_RING):
        pltpu.async_copy(msg_hbm.at[gidx(b)], rows_v.at[b], sems[b])

    def step(s, carry):
        for b in range(GS_RING):
            c = GS_RING * s + b
            pltpu.make_async_copy(
                msg_hbm.at[gidx(c)], rows_v.at[b], sems[b]).wait()
            for i in range(2):
                for j in range(8):
                    acc = rows_v[b, 32 * i, pl.ds(16 * j, 16)]
                    for k in range(1, 32):
                        acc = acc + rows_v[b, 32 * i + k, pl.ds(16 * j, 16)]
                    out_v[pl.ds((2 * c + i) * H + 16 * j, 16)] = acc
            pltpu.async_copy(
                msg_hbm.at[gidx(c + GS_RING)], rows_v.at[b], sems[b])
        return carry

    lax.fori_loop(0, GS_CHUNKS // GS_RING, step, 0)
    for b in range(GS_RING):
        pltpu.make_async_copy(
            msg_hbm.at[gidx(b)], rows_v.at[b], sems[b]).wait()
    pltpu.sync_copy(out_v, amsg_hbm.at[pl.ds(wid * 320 * H, 320 * H)])


def _sc_gather_sum(message, a2b_r):
    out_flat = pl.kernel(
        _gs_body,
        out_type=jax.ShapeDtypeStruct((NAP * H,), jnp.float32),
        mesh=plsc.VectorSubcoreMesh(core_axis_name="c", subcore_axis_name="s"),
        scratch_types=[
            pltpu.VMEM(((GS_CHUNKS + GS_RING) * GS_CHUNK,), jnp.int32),
            pltpu.VMEM((GS_RING, GS_CHUNK, H), jnp.float32),
            pltpu.VMEM((320 * H,), jnp.float32),
        ] + [pltpu.SemaphoreType.DMA] * GS_RING,
    )(message, a2b_r)
    return out_flat.reshape(NAP, H)


def _edge_body(amsg_hbm, msg_hbm, b2a_hbm, b2r_hbm, t_hbm, dummy_hbm,
               idxa_v, idxr_v, ga_v, gr_v, to_v, *sems):
    wid = lax.axis_index("s") * NC + lax.axis_index("c")
    pltpu.sync_copy(b2a_hbm.at[wid], idxa_v)
    pltpu.sync_copy(b2r_hbm.at[wid], idxr_v)

    def eidx(iv, c):
        return iv.at[pl.ds(c * ED_CHUNK, ED_CHUNK)]
    sas = sems[:ED_RING]
    srs = sems[ED_RING:2 * ED_RING]
    sos = sems[2 * ED_RING:]
    base = wid * 10000
    for b in range(ED_RING):
        pltpu.async_copy(amsg_hbm.at[eidx(idxa_v, b)], ga_v.at[b], sas[b])
        pltpu.async_copy(msg_hbm.at[eidx(idxr_v, b)], gr_v.at[b], srs[b])
        pltpu.async_copy(to_v.at[b], dummy_hbm.at[wid], sos[b])

    def step(s, carry):
        for b in range(ED_RING):
            c = ED_RING * s + b
            pltpu.make_async_copy(
                amsg_hbm.at[eidx(idxa_v, c)], ga_v.at[b], sas[b]).wait()
            pltpu.make_async_copy(
                msg_hbm.at[eidx(idxr_v, c)], gr_v.at[b], srs[b]).wait()
            pltpu.make_async_copy(to_v.at[b], dummy_hbm.at[wid], sos[b]).wait()
            for r in range(ED_CHUNK):
                for j in range(8):
                    to_v[b, r, pl.ds(16 * j, 16)] = (
                        ga_v[b, r, pl.ds(16 * j, 16)]
                        - gr_v[b, r, pl.ds(16 * j, 16)])
            pltpu.async_copy(
                to_v.at[b],
                t_hbm.at[pl.ds(base + c * ED_CHUNK, ED_CHUNK)], sos[b])
            pltpu.async_copy(
                amsg_hbm.at[eidx(idxa_v, c + ED_RING)], ga_v.at[b], sas[b])
            pltpu.async_copy(
                msg_hbm.at[eidx(idxr_v, c + ED_RING)], gr_v.at[b], srs[b])
        return carry

    lax.fori_loop(0, ED_CHUNKS // ED_RING, step, 0)
    for b in range(ED_RING):
        pltpu.make_async_copy(
            amsg_hbm.at[eidx(idxa_v, b)], ga_v.at[b], sas[b]).wait()
        pltpu.make_async_copy(
            msg_hbm.at[eidx(idxr_v, b)], gr_v.at[b], srs[b]).wait()
        pltpu.make_async_copy(to_v.at[b], dummy_hbm.at[wid], sos[b]).wait()


def _sc_edge(a_msg, message, b2a_r, b2r_r):
    t, _ = pl.kernel(
        _edge_body,
        out_type=[
            jax.ShapeDtypeStruct((NB, H), jnp.float32),
            jax.ShapeDtypeStruct((NW, ED_CHUNK, H), jnp.float32),
        ],
        mesh=plsc.VectorSubcoreMesh(core_axis_name="c", subcore_axis_name="s"),
        scratch_types=[
            pltpu.VMEM(((ED_CHUNKS + ED_RING) * ED_CHUNK,), jnp.int32),
            pltpu.VMEM(((ED_CHUNKS + ED_RING) * ED_CHUNK,), jnp.int32),
            pltpu.VMEM((ED_RING, ED_CHUNK, H), jnp.float32),
            pltpu.VMEM((ED_RING, ED_CHUNK, H), jnp.float32),
            pltpu.VMEM((ED_RING, ED_CHUNK, H), jnp.float32),
        ] + [pltpu.SemaphoreType.DMA] * (3 * ED_RING),
    )(a_msg, message, b2a_r, b2r_r)
    return t


def kernel(f_atoms, f_bonds, a2b, b2a, b2revb, mask, W_i, W_h, W_o):
    a2b = a2b.astype(jnp.int32)
    b2a = b2a.astype(jnp.int32)
    b2revb = b2revb.astype(jnp.int32)

    a2b_pad = jnp.zeros((NAP, MAX_NB), jnp.int32).at[:NA].set(a2b)
    a2b_r = jnp.pad(a2b_pad.reshape(NW, GS_CHUNKS * GS_CHUNK),
                    ((0, 0), (0, GS_RING * GS_CHUNK)))
    b2a_r = jnp.pad(b2a.reshape(NW, ED_CHUNKS * ED_CHUNK),
                    ((0, 0), (0, ED_RING * ED_CHUNK)))
    b2r_r = jnp.pad(b2revb.reshape(NW, ED_CHUNKS * ED_CHUNK),
                    ((0, 0), (0, ED_RING * ED_CHUNK)))

    message = _init_mm(f_bonds, W_i)
    for _ in range(DEPTH - 1):
        a_msg = _sc_gather_sum(message, a2b_r)
        t = _sc_edge(a_msg, message, b2a_r, b2r_r)
        message = _layer_mm(f_bonds, t, W_i, W_h)
    a_msg = _sc_gather_sum(message, a2b_r)
    return _final_mm(f_atoms, a_msg[:NA], W_o, mask)
